# Initial kernel scaffold; baseline (speedup 1.0000x reference)
import jax
import jax.numpy as jnp
from jax.experimental import pallas as pl


def kernel(x, token_table, pos_table):
    raise NotImplementedError("write your pallas kernel here")


# speedup vs baseline: 4.6498x; 4.6498x over previous
import functools

import jax
import jax.numpy as jnp
from jax import lax
from jax.experimental import pallas as pl
from jax.experimental.pallas import tpu as pltpu
from jax.experimental.pallas import tpu_sc as plsc

D_MODEL = 128
CHUNK = 128


def _add_kernel(a_ref, b_ref, o_ref):
    o_ref[...] = a_ref[...] + b_ref[...]


def _sum_tables(a, b):
    n, d = a.shape
    blk = 2000
    grid = n // blk
    return pl.pallas_call(
        _add_kernel,
        out_shape=jax.ShapeDtypeStruct((n, d), a.dtype),
        grid=(grid,),
        in_specs=[
            pl.BlockSpec((blk, d), lambda i: (i, 0)),
            pl.BlockSpec((blk, d), lambda i: (i, 0)),
        ],
        out_specs=pl.BlockSpec((blk, d), lambda i: (i, 0)),
    )(a, b)


@functools.lru_cache(maxsize=None)
def _make_gather(n_chunks_total, chunk, d):
    info = plsc.get_sparse_core_info()
    nc, ns = info.num_cores, info.num_subcores
    nw = nc * ns
    per_w = n_chunks_total // nw

    mesh = plsc.VectorSubcoreMesh(core_axis_name="c", subcore_axis_name="s")

    @functools.partial(
        pl.kernel,
        out_type=jax.ShapeDtypeStruct((n_chunks_total * chunk, d), jnp.float32),
        mesh=mesh,
        scratch_types=[
            pltpu.VMEM((per_w, chunk), jnp.int32),
            pltpu.VMEM((chunk, d), jnp.float32),
            pltpu.SemaphoreType.DMA,
        ],
    )
    def gather_kernel(x_hbm, tab_hbm, out_hbm, idx_v, buf, sem):
        wid = lax.axis_index("s") * nc + lax.axis_index("c")
        chunk0 = wid * per_w
        pltpu.sync_copy(x_hbm.at[wid], idx_v)

        def body(j, carry):
            pltpu.async_copy(tab_hbm.at[idx_v.at[j]], buf, sem).wait()
            pltpu.sync_copy(
                buf, out_hbm.at[pl.ds((chunk0 + j) * chunk, chunk)]
            )
            return carry

        lax.fori_loop(0, per_w, body, 0)

    return gather_kernel


def kernel(x, token_table, pos_table):
    summed = _sum_tables(token_table, pos_table)
    b, s = x.shape
    n = b * s
    gather = _make_gather(n // CHUNK, CHUNK, D_MODEL)
    nw = plsc.get_sparse_core_info().num_cores * plsc.get_sparse_core_info().num_subcores
    x3d = x.reshape(nw, n // CHUNK // nw, CHUNK).astype(jnp.int32)
    out = gather(x3d, summed)
    return out.reshape(b, s, D_MODEL)
